# baseline (speedup 1.0000x reference)
import functools
import jax
import jax.numpy as jnp
from jax import lax
from jax.experimental import pallas as pl
from jax.experimental.pallas import tpu as pltpu
from jax.experimental.pallas import tpu_sc as plsc

CHUNK = 256
NSLOTS = 2


def kernel(x, z, idx_pa):
    B, N, DX = x.shape
    DZ = z.shape[2]
    S = idx_pa.shape[1]
    idx = idx_pa.astype(jnp.int32)

    mesh = plsc.VectorSubcoreMesh(core_axis_name="c", subcore_axis_name="s")
    NCORE, NSUB = 2, 16
    PHASES = B // NCORE
    TS = S // NSUB
    n_chunks = TS // CHUNK

    @functools.partial(
        pl.kernel,
        mesh=mesh,
        compiler_params=pltpu.CompilerParams(use_tc_tiling_on_sc=False),
        out_type=(
            jax.ShapeDtypeStruct((B, S, DX), jnp.float32),
            jax.ShapeDtypeStruct((B, S, DZ), jnp.float32),
        ),
        scratch_types=[
            pltpu.VMEM_SHARED((N, DX), jnp.float32),
            pltpu.VMEM_SHARED((N, DZ), jnp.float32),
            pltpu.VMEM((TS,), jnp.int32),
            pltpu.VMEM((NSLOTS, CHUNK, DX), jnp.float32),
            pltpu.VMEM((NSLOTS, CHUNK, DZ), jnp.float32),
            pltpu.SemaphoreType.DMA((NSLOTS,)),
            pltpu.SemaphoreType.DMA((NSLOTS,)),
            pltpu.SemaphoreType.DMA,
            pltpu.SemaphoreType.DMA,
        ],
    )
    def gather_kernel(x_hbm, z_hbm, idx_hbm, ox_hbm, oz_hbm,
                      sx, sz, idx_v, xrows_v, zrows_v,
                      gsem, wsem, ssem, isem):
        c = lax.axis_index("c")
        t = lax.axis_index("s")

        def stage_start(b):
            pltpu.async_copy(x_hbm.at[b], sx, ssem)
            pltpu.async_copy(z_hbm.at[b], sz, ssem)

        def stage_wait(b):
            pltpu.make_async_copy(x_hbm.at[b], sx, ssem).wait()
            pltpu.make_async_copy(z_hbm.at[b], sz, ssem).wait()

        def gather_start(j, slot):
            ids = idx_v.at[pl.ds(j * CHUNK, CHUNK)]
            pltpu.async_copy(sx.at[ids], xrows_v.at[slot], gsem.at[slot])
            pltpu.async_copy(sz.at[ids], zrows_v.at[slot], gsem.at[slot])

        def gather_wait(j, slot):
            ids = idx_v.at[pl.ds(j * CHUNK, CHUNK)]
            pltpu.make_async_copy(sx.at[ids], xrows_v.at[slot],
                                  gsem.at[slot]).wait()
            pltpu.make_async_copy(sz.at[ids], zrows_v.at[slot],
                                  gsem.at[slot]).wait()

        def wb_start(b, j, slot):
            base = t * TS + j * CHUNK
            pltpu.async_copy(xrows_v.at[slot],
                             ox_hbm.at[b, pl.ds(base, CHUNK)], wsem.at[slot])
            pltpu.async_copy(zrows_v.at[slot],
                             oz_hbm.at[b, pl.ds(base, CHUNK)], wsem.at[slot])

        def wb_wait(b, j, slot):
            base = t * TS + j * CHUNK
            pltpu.make_async_copy(xrows_v.at[slot],
                                  ox_hbm.at[b, pl.ds(base, CHUNK)],
                                  wsem.at[slot]).wait()
            pltpu.make_async_copy(zrows_v.at[slot],
                                  oz_hbm.at[b, pl.ds(base, CHUNK)],
                                  wsem.at[slot]).wait()

        @pl.when(t == 0)
        def _():
            stage_start(c * PHASES)

        @pl.loop(0, PHASES)
        def _(p):
            b = c * PHASES + p
            pltpu.async_copy(idx_hbm.at[b, pl.ds(t * TS, TS)], idx_v,
                             isem).wait()

            @pl.when(t == 0)
            def _():
                stage_wait(b)

            plsc.subcore_barrier()

            for j in range(min(NSLOTS, n_chunks)):
                gather_start(j, j % NSLOTS)
            for j in range(n_chunks):
                slot = j % NSLOTS
                gather_wait(j, slot)
                wb_start(b, j, slot)
                if j + NSLOTS < n_chunks:
                    wb_wait(b, j, slot)
                    gather_start(j + NSLOTS, slot)
            for j in range(max(0, n_chunks - NSLOTS), n_chunks):
                wb_wait(b, j, j % NSLOTS)

            plsc.subcore_barrier()

            @pl.when(jnp.logical_and(t == 0, p + 1 < PHASES))
            def _():
                stage_start(b + 1)

    return gather_kernel(x, z, idx)


# speedup vs baseline: 1.0329x; 1.0329x over previous
import functools
import jax
import jax.numpy as jnp
from jax import lax
from jax.experimental import pallas as pl
from jax.experimental.pallas import tpu as pltpu
from jax.experimental.pallas import tpu_sc as plsc

CHUNK = 256
NSLOTS = 4


def kernel(x, z, idx_pa):
    B, N, DX = x.shape
    DZ = z.shape[2]
    S = idx_pa.shape[1]
    idx = idx_pa.astype(jnp.int32)
    n_chunks = S // CHUNK

    mesh = plsc.VectorSubcoreMesh(core_axis_name="c", subcore_axis_name="s")

    @functools.partial(
        pl.kernel,
        mesh=mesh,
        compiler_params=pltpu.CompilerParams(use_tc_tiling_on_sc=False),
        out_type=(
            jax.ShapeDtypeStruct((B, S, DX), jnp.float32),
            jax.ShapeDtypeStruct((B, S, DZ), jnp.float32),
        ),
        scratch_types=[
            pltpu.VMEM((S,), jnp.int32),
            pltpu.VMEM((NSLOTS, CHUNK, DX), jnp.float32),
            pltpu.VMEM((NSLOTS, CHUNK, DZ), jnp.float32),
            pltpu.SemaphoreType.DMA((NSLOTS,)),
            pltpu.SemaphoreType.DMA((NSLOTS,)),
            pltpu.SemaphoreType.DMA,
        ],
    )
    def gather_kernel(x_hbm, z_hbm, idx_hbm, ox_hbm, oz_hbm,
                      idx_v, xrows_v, zrows_v, gsem, wsem, isem):
        w = lax.axis_index("s") * 2 + lax.axis_index("c")
        pltpu.async_copy(idx_hbm.at[w], idx_v, isem).wait()

        def gather_start(c, slot):
            pltpu.async_copy(
                x_hbm.at[w].at[idx_v.at[pl.ds(c * CHUNK, CHUNK)]],
                xrows_v.at[slot], gsem.at[slot])
            pltpu.async_copy(
                z_hbm.at[w].at[idx_v.at[pl.ds(c * CHUNK, CHUNK)]],
                zrows_v.at[slot], gsem.at[slot])

        def gather_wait(c, slot):
            pltpu.make_async_copy(
                x_hbm.at[w].at[idx_v.at[pl.ds(c * CHUNK, CHUNK)]],
                xrows_v.at[slot], gsem.at[slot]).wait()
            pltpu.make_async_copy(
                z_hbm.at[w].at[idx_v.at[pl.ds(c * CHUNK, CHUNK)]],
                zrows_v.at[slot], gsem.at[slot]).wait()

        def wb_start(c, slot):
            pltpu.async_copy(
                xrows_v.at[slot], ox_hbm.at[w, pl.ds(c * CHUNK, CHUNK)],
                wsem.at[slot])
            pltpu.async_copy(
                zrows_v.at[slot], oz_hbm.at[w, pl.ds(c * CHUNK, CHUNK)],
                wsem.at[slot])

        def wb_wait(c, slot):
            pltpu.make_async_copy(
                xrows_v.at[slot], ox_hbm.at[w, pl.ds(c * CHUNK, CHUNK)],
                wsem.at[slot]).wait()
            pltpu.make_async_copy(
                zrows_v.at[slot], oz_hbm.at[w, pl.ds(c * CHUNK, CHUNK)],
                wsem.at[slot]).wait()

        for k in range(NSLOTS):
            gather_start(k, k)

        @pl.loop(0, n_chunks - NSLOTS, step=NSLOTS)
        def _(c):
            for k in range(NSLOTS):
                gather_wait(c + k, k)
                wb_start(c + k, k)
            for k in range(NSLOTS):
                wb_wait(c + k, k)
                gather_start(c + NSLOTS + k, k)

        c = n_chunks - NSLOTS
        for k in range(NSLOTS):
            gather_wait(c + k, k)
            wb_start(c + k, k)
        for k in range(NSLOTS):
            wb_wait(c + k, k)

    return gather_kernel(x, z, idx)
